# Initial kernel scaffold; baseline (speedup 1.0000x reference)
import functools

import jax
import jax.numpy as jnp
from jax import lax
from jax.experimental import pallas as pl
from jax.experimental.pallas import tpu as pltpu
from jax.experimental.pallas import tpu_sc as plsc

N = 10000; E = 320000; B = 100; NPG = 100
DIN = 128; DGH = 128; DG = 64; DH = 32; RANK = 16; K = 4; FL = 49; DESC = 196

NC = 2
NS = 16
CHUNK = 128
NCHUNK = E // CHUNK
CPW = -(-NCHUNK // (NC * NS))
RPS = N // NS

_f32 = jnp.float32


def _sc_mesh():
    return plsc.VectorSubcoreMesh(core_axis_name="c", subcore_axis_name="s")


def _deg_body(dst_hbm, zeros_hbm, out_hbm, dstv, onesv, acc_sh, gsem):
    c = lax.axis_index("c")
    s = lax.axis_index("s")
    w = s * NC + c

    for k in range(CHUNK // 16):
        onesv[pl.ds(k * 16, 16)] = jnp.ones((16,), _f32)

    @pl.when(s == 0)
    def _():
        pltpu.sync_copy(zeros_hbm, acc_sh)

    plsc.subcore_barrier()

    def step(j, carry):
        ci = j * (NC * NS) + w

        @pl.when(ci < NCHUNK)
        def _():
            pltpu.async_copy(dst_hbm.at[ci], dstv, gsem).wait()
            pltpu.sync_copy(onesv, acc_sh.at[dstv], add=True)

        return carry

    lax.fori_loop(0, CPW, step, 0)
    plsc.subcore_barrier()

    @pl.when(s == 0)
    def _():
        pltpu.sync_copy(acc_sh, out_hbm.at[c])


def _sc_degree(dst2d, zeros1d):
    return pl.kernel(
        _deg_body,
        out_type=jax.ShapeDtypeStruct((NC, N), _f32),
        mesh=_sc_mesh(),
        scratch_types=[
            pltpu.VMEM((CHUNK,), jnp.int32),
            pltpu.VMEM((CHUNK,), _f32),
            pltpu.VMEM_SHARED((N,), _f32),
            pltpu.SemaphoreType.DMA,
        ],
    )(dst2d, zeros1d)


def _scatter_body(D, msg_hbm, src_hbm, dst_hbm, zeros_hbm, out_hbm,
                  srcv, dstv, rows, acc_sh, gsem, isem):
    c = lax.axis_index("c")
    s = lax.axis_index("s")
    w = s * NC + c

    pltpu.sync_copy(zeros_hbm.at[pl.ds(s * RPS, RPS)],
                    acc_sh.at[pl.ds(s * RPS, RPS)])
    plsc.subcore_barrier()

    def step(j, carry):
        ci = j * (NC * NS) + w

        @pl.when(ci < NCHUNK)
        def _():
            cp1 = pltpu.async_copy(src_hbm.at[ci], srcv, isem)
            cp2 = pltpu.async_copy(dst_hbm.at[ci], dstv, isem)
            cp1.wait()
            cp2.wait()
            pltpu.async_copy(msg_hbm.at[srcv], rows, gsem).wait()
            pltpu.sync_copy(rows, acc_sh.at[dstv], add=True)

        return carry

    lax.fori_loop(0, CPW, step, 0)
    plsc.subcore_barrier()
    pltpu.sync_copy(acc_sh.at[pl.ds(s * RPS, RPS)],
                    out_hbm.at[c, pl.ds(s * RPS, RPS)])


def _sc_scatter(msg, src2d, dst2d, zeros2d, D):
    body = functools.partial(_scatter_body, D)
    return pl.kernel(
        body,
        out_type=jax.ShapeDtypeStruct((NC, N, D), _f32),
        mesh=_sc_mesh(),
        scratch_types=[
            pltpu.VMEM((CHUNK,), jnp.int32),
            pltpu.VMEM((CHUNK,), jnp.int32),
            pltpu.VMEM((CHUNK, D), _f32),
            pltpu.VMEM_SHARED((N, D), _f32),
            pltpu.SemaphoreType.DMA,
            pltpu.SemaphoreType.DMA,
        ],
    )(msg, src2d, dst2d, zeros2d)


RB = 1000
NG = N // RB


def _t1_body(deg_ref, x_ref, w1_ref, m1_ref, dis_ref):
    deg = deg_ref[0, 0, 0, :] + deg_ref[1, 0, 0, :] + 1.0
    dis = lax.rsqrt(deg)
    m = jnp.dot(x_ref[...], w1_ref[...], preferred_element_type=_f32)
    m1_ref[...] = m * dis[:, None]
    dis_ref[0, 0, :] = dis


def _tc1(deg4d, x, W1):
    return pl.pallas_call(
        _t1_body,
        grid=(NG,),
        in_specs=[
            pl.BlockSpec((2, 1, 1, RB), lambda i: (0, i, 0, 0)),
            pl.BlockSpec((RB, DIN), lambda i: (i, 0)),
            pl.BlockSpec((DIN, DGH), lambda i: (0, 0)),
        ],
        out_specs=[
            pl.BlockSpec((RB, DGH), lambda i: (i, 0)),
            pl.BlockSpec((1, 1, RB), lambda i: (i, 0, 0)),
        ],
        out_shape=[
            jax.ShapeDtypeStruct((N, DGH), _f32),
            jax.ShapeDtypeStruct((NG, 1, RB), _f32),
        ],
    )(deg4d, x, W1)


def _t2_body(agg_ref, m1_ref, dis_ref, w2_ref, m2_ref):
    dis = dis_ref[0, 0, :]
    a = agg_ref[0] + agg_ref[1] + m1_ref[...]
    out1 = jnp.maximum(a * dis[:, None], 0.0)
    m2_ref[...] = jnp.dot(out1, w2_ref[...], preferred_element_type=_f32) \
        * dis[:, None]


def _tc2(agg1, m1, dis3d, W2):
    return pl.pallas_call(
        _t2_body,
        grid=(NG,),
        in_specs=[
            pl.BlockSpec((2, RB, DGH), lambda i: (0, i, 0)),
            pl.BlockSpec((RB, DGH), lambda i: (i, 0)),
            pl.BlockSpec((1, 1, RB), lambda i: (i, 0, 0)),
            pl.BlockSpec((DGH, DG), lambda i: (0, 0)),
        ],
        out_specs=pl.BlockSpec((RB, DG), lambda i: (i, 0)),
        out_shape=jax.ShapeDtypeStruct((N, DG), _f32),
    )(agg1, m1, dis3d, W2)


def _t3_body(agg_ref, m2_ref, dis_ref, desc_ref, logscale_ref, wm, out_ref,
             beta_ref, alpha_ref):
    eps = 1e-5
    Hf = agg_ref[0] + agg_ref[1] + m2_ref[...]
    dis2 = dis_ref[...]
    H3 = Hf.reshape(B, NPG, DG) * dis2[:, :, None]
    hg = jnp.mean(H3, axis=1)
    Hs = H3.reshape(N, DG)

    def ln32(t, g, b):
        mu = jnp.dot(t, wm['Gavg'], preferred_element_type=_f32)
        xm = t - mu
        var = jnp.dot(xm * xm, wm['Gavg'], preferred_element_type=_f32)
        return xm * lax.rsqrt(var + eps) * g + b

    tokpre = jnp.dot(desc_ref[...], wm['TokW'], preferred_element_type=_f32) \
        + wm['tokb']
    tok = jnp.maximum(ln32(tokpre, wm['tokg'], wm['tokB']), 0.0)

    kk = jnp.dot(Hs, wm['WkT'], preferred_element_type=_f32)
    v = jnp.dot(Hs, wm['WvT'], preferred_element_type=_f32)
    qf = jnp.dot(tok, wm['WqT'], preferred_element_type=_f32)
    qexp = jnp.broadcast_to(qf[:, None, :], (B, NPG, K * RANK)) \
        .reshape(N, K * RANK)
    scale = jnp.maximum(jnp.exp(logscale_ref[...]), 0.1)
    scores = jnp.dot(kk * qexp, wm['Gsum16'], preferred_element_type=_f32)
    scores3 = scores.reshape(B, NPG, K) / scale[0][None, None, :]
    mx = jnp.max(scores3, axis=1, keepdims=True)
    ex = jnp.exp(scores3 - mx)
    den = jnp.sum(ex, axis=1, keepdims=True)
    alpha3 = ex / den
    aexp = jnp.dot(alpha3.reshape(N, K), wm['Gexp32'],
                   preferred_element_type=_f32)
    ctx = jnp.sum((aexp * v).reshape(B, NPG, K * DH), axis=1)

    fused = (jnp.dot(ctx, wm['P0'], preferred_element_type=_f32)
             + jnp.dot(tok, wm['P1'], preferred_element_type=_f32)
             + jnp.dot(ctx * tok, wm['P2'], preferred_element_type=_f32)
             + jnp.dot(ctx - tok, wm['P3'], preferred_element_type=_f32))
    f1 = jnp.dot(fused, wm['FW1'], preferred_element_type=_f32) + wm['fb1']
    f1 = jnp.maximum(ln32(f1, wm['fg'], wm['fB']), 0.0)
    reps = jnp.dot(f1, wm['FW2'], preferred_element_type=_f32) + wm['fb2']

    gp = jnp.dot(hg, wm['agWT'], preferred_element_type=_f32) + wm['agb']
    fp = jnp.dot(reps, wm['AfT'], preferred_element_type=_f32) + wm['afb']
    gpexp = jnp.dot(gp, wm['Gtile32'], preferred_element_type=_f32)
    t = jnp.tanh(fp + gpexp)
    sc = jnp.dot(t, wm['AsT'], preferred_element_type=_f32) + wm['asb']
    mxs = jnp.max(sc, axis=1, keepdims=True)
    exs = jnp.exp(sc - mxs)
    beta = exs / jnp.sum(exs, axis=1, keepdims=True)
    bexp = jnp.dot(beta, wm['Gexp32'], preferred_element_type=_f32)
    h_fam = jnp.dot(bexp * reps, wm['Gfold'], preferred_element_type=_f32)

    z = (jnp.dot(hg, wm['Pz1'], preferred_element_type=_f32)
         + jnp.dot(h_fam, wm['Pz2'], preferred_element_type=_f32))
    z = jnp.maximum(jnp.dot(z, wm['hW1T'], preferred_element_type=_f32)
                    + wm['hb1'], 0.0)
    z = jnp.maximum(jnp.dot(z, wm['hW2T'], preferred_element_type=_f32)
                    + wm['hb2'], 0.0)
    out_ref[...] = jnp.dot(z, wm['hW3T'], preferred_element_type=_f32) \
        + wm['hb3']
    beta_ref[...] = beta
    alpha_ref[...] = jnp.swapaxes(alpha3, 1, 2)


def _tc3(agg2, m2, dis2d, desc, logscale2d, wmats):
    names = sorted(wmats.keys())
    vals = [wmats[k] for k in names]

    def body(agg_ref, m2_ref, dis_ref, desc_ref, ls_ref, *rest):
        wrefs = {k: r[...] for k, r in zip(names, rest[:len(names)])}
        out_ref, beta_ref, alpha_ref = rest[len(names):]
        _t3_body(agg_ref, m2_ref, dis_ref, desc_ref, ls_ref, wrefs,
                 out_ref, beta_ref, alpha_ref)

    return pl.pallas_call(
        body,
        out_shape=[
            jax.ShapeDtypeStruct((B, 1), _f32),
            jax.ShapeDtypeStruct((B, K), _f32),
            jax.ShapeDtypeStruct((B, K, NPG), _f32),
        ],
    )(agg2, m2, dis2d, desc, logscale2d, *vals)


def _prep_weights(p):
    bd = jax.scipy.linalg.block_diag
    i128 = jnp.arange(128)
    wm = {}
    wm['TokW'] = bd(*[p['tokW'][k].T for k in range(K)])
    wm['tokb'] = p['tokb'].reshape(-1)
    wm['tokg'] = p['tokg'].reshape(-1)
    wm['tokB'] = p['tokB'].reshape(-1)
    wm['Gavg'] = ((i128[:, None] // DH == i128[None, :] // DH)
                  .astype(_f32) / DH)
    wm['WkT'] = p['Wk'].transpose(2, 0, 1).reshape(DG, K * RANK)
    wm['WvT'] = p['Wv'].transpose(2, 0, 1).reshape(DG, K * DH)
    wm['WqT'] = bd(*[p['Wq'][k].T for k in range(K)])
    wm['Gsum16'] = (jnp.arange(K * RANK)[:, None] // RANK
                    == jnp.arange(K)[None, :]).astype(_f32)
    wm['Gexp32'] = (jnp.arange(K)[:, None]
                    == i128[None, :] // DH).astype(_f32)
    for pp in range(4):
        wm[f'P{pp}'] = (jnp.arange(K * 4 * DH)[None, :]
                        == (i128[:, None] // DH) * (4 * DH)
                        + pp * DH + i128[:, None] % DH).astype(_f32)
    wm['FW1'] = bd(*[p['fW1'][k].T for k in range(K)])
    wm['fb1'] = p['fb1'].reshape(-1)
    wm['fg'] = p['fg'].reshape(-1)
    wm['fB'] = p['fB'].reshape(-1)
    wm['FW2'] = bd(*[p['fW2'][k].T for k in range(K)])
    wm['fb2'] = p['fb2'].reshape(-1)
    wm['agWT'] = p['agW'].T
    wm['agb'] = p['agb']
    wm['AfT'] = bd(*[p['afW'].T] * K)
    wm['afb'] = jnp.tile(p['afb'], K)
    wm['Gtile32'] = (jnp.arange(DH)[:, None]
                     == i128[None, :] % DH).astype(_f32)
    wm['AsT'] = bd(*[p['asW'].T] * K)
    wm['asb'] = p['asb']
    wm['Gfold'] = (i128[:, None] % DH
                   == jnp.arange(DH)[None, :]).astype(_f32)
    wm['Pz1'] = jnp.concatenate([jnp.eye(DG, dtype=_f32),
                                 jnp.zeros((DG, DH), _f32)], axis=1)
    wm['Pz2'] = jnp.concatenate([jnp.zeros((DH, DG), _f32),
                                 jnp.eye(DH, dtype=_f32)], axis=1)
    wm['hW1T'] = p['hW1'].T
    wm['hb1'] = p['hb1']
    wm['hW2T'] = p['hW2'].T
    wm['hb2'] = p['hb2']
    wm['hW3T'] = p['hW3'].T
    wm['hb3'] = p['hb3']
    return wm


def kernel(x, edge_index, desc, params):
    p = params
    src2d = edge_index[0].astype(jnp.int32).reshape(NCHUNK, CHUNK)
    dst2d = edge_index[1].astype(jnp.int32).reshape(NCHUNK, CHUNK)
    zeros1d = jnp.zeros((N,), _f32)
    zerosA = jnp.zeros((N, DGH), _f32)
    zerosB = jnp.zeros((N, DG), _f32)

    deg_parts = _sc_degree(dst2d, zeros1d)
    deg4d = deg_parts.reshape(2, NG, 1, RB)

    m1, dis3d = _tc1(deg4d, x, p['W1'])
    agg1 = _sc_scatter(m1, src2d, dst2d, zerosA, DGH)
    m2 = _tc2(agg1, m1, dis3d, p['W2'])
    agg2 = _sc_scatter(m2, src2d, dst2d, zerosB, DG)

    dis2d = dis3d.reshape(B, NPG)
    wm = _prep_weights(p)
    out, beta, alpha = _tc3(agg2, m2, dis2d, desc,
                            p['logscale'].reshape(1, K), wm)
    return out, beta, alpha


# speedup vs baseline: 5.0068x; 5.0068x over previous
import functools

import jax
import jax.numpy as jnp
from jax import lax
from jax.experimental import pallas as pl
from jax.experimental.pallas import tpu as pltpu
from jax.experimental.pallas import tpu_sc as plsc

N = 10000; E = 320000; B = 100; NPG = 100
DIN = 128; DGH = 128; DG = 64; DH = 32; RANK = 16; K = 4; FL = 49; DESC = 196

NC = 2
NS = 16
CHUNK = 128
NCHUNK = E // CHUNK
CPW = -(-NCHUNK // (NC * NS))
RPS = 640
RPS_LAST = N - RPS * (NS - 1)

_f32 = jnp.float32


def _sc_mesh():
    return plsc.VectorSubcoreMesh(core_axis_name="c", subcore_axis_name="s")


DW = 128


def _deg_body(dst_hbm, zeros_hbm, ones_hbm, out_hbm, dstv, onesv, acc_sh,
              gsem):
    c = lax.axis_index("c")
    s = lax.axis_index("s")
    w = s * NC + c

    pltpu.sync_copy(ones_hbm, onesv)

    @pl.when(s < NS - 1)
    def _():
        pltpu.sync_copy(zeros_hbm.at[pl.ds(s * RPS, RPS)],
                        acc_sh.at[pl.ds(s * RPS, RPS)])

    @pl.when(s == NS - 1)
    def _():
        pltpu.sync_copy(zeros_hbm.at[pl.ds((NS - 1) * RPS, RPS_LAST)],
                        acc_sh.at[pl.ds((NS - 1) * RPS, RPS_LAST)])

    plsc.subcore_barrier()

    def step(j, carry):
        ci = j * (NC * NS) + w

        @pl.when(ci < NCHUNK)
        def _():
            pltpu.async_copy(dst_hbm.at[pl.ds(ci * CHUNK, CHUNK)], dstv,
                             gsem).wait()
            pltpu.sync_copy(onesv, acc_sh.at[dstv], add=True)

        return carry

    lax.fori_loop(0, CPW, step, 0)
    plsc.subcore_barrier()

    @pl.when(s < NS - 1)
    def _():
        pltpu.sync_copy(acc_sh.at[pl.ds(s * RPS, RPS)],
                        out_hbm.at[c, pl.ds(s * RPS, RPS)])

    @pl.when(s == NS - 1)
    def _():
        pltpu.sync_copy(acc_sh.at[pl.ds((NS - 1) * RPS, RPS_LAST)],
                        out_hbm.at[c, pl.ds((NS - 1) * RPS, RPS_LAST)])


def _sc_degree(dst1d, zeros2d, ones2d):
    return pl.kernel(
        _deg_body,
        out_type=jax.ShapeDtypeStruct((NC, N, DW), _f32),
        mesh=_sc_mesh(),
        scratch_types=[
            pltpu.VMEM((CHUNK,), jnp.int32),
            pltpu.VMEM((CHUNK, DW), _f32),
            pltpu.VMEM_SHARED((N, DW), _f32),
            pltpu.SemaphoreType.DMA,
        ],
    )(dst1d, zeros2d, ones2d)


def _scatter_body(D, msg_hbm, src_hbm, dst_hbm, zeros_hbm, out_hbm,
                  srcv, dstv, rows, acc_sh, gsem, isem):
    c = lax.axis_index("c")
    s = lax.axis_index("s")
    w = s * NC + c

    @pl.when(s < NS - 1)
    def _():
        pltpu.sync_copy(zeros_hbm.at[pl.ds(s * RPS, RPS)],
                        acc_sh.at[pl.ds(s * RPS, RPS)])

    @pl.when(s == NS - 1)
    def _():
        pltpu.sync_copy(zeros_hbm.at[pl.ds((NS - 1) * RPS, RPS_LAST)],
                        acc_sh.at[pl.ds((NS - 1) * RPS, RPS_LAST)])

    plsc.subcore_barrier()

    def step(j, carry):
        ci = j * (NC * NS) + w

        @pl.when(ci < NCHUNK)
        def _():
            cp1 = pltpu.async_copy(src_hbm.at[pl.ds(ci * CHUNK, CHUNK)],
                                   srcv, isem)
            cp2 = pltpu.async_copy(dst_hbm.at[pl.ds(ci * CHUNK, CHUNK)],
                                   dstv, isem)
            cp1.wait()
            cp2.wait()
            pltpu.async_copy(msg_hbm.at[srcv], rows, gsem).wait()
            pltpu.sync_copy(rows, acc_sh.at[dstv], add=True)

        return carry

    lax.fori_loop(0, CPW, step, 0)
    plsc.subcore_barrier()

    @pl.when(s < NS - 1)
    def _():
        pltpu.sync_copy(acc_sh.at[pl.ds(s * RPS, RPS)],
                        out_hbm.at[c, pl.ds(s * RPS, RPS)])

    @pl.when(s == NS - 1)
    def _():
        pltpu.sync_copy(acc_sh.at[pl.ds((NS - 1) * RPS, RPS_LAST)],
                        out_hbm.at[c, pl.ds((NS - 1) * RPS, RPS_LAST)])


def _sc_scatter(msg, src1d, dst1d, zeros2d, D):
    body = functools.partial(_scatter_body, D)
    return pl.kernel(
        body,
        out_type=jax.ShapeDtypeStruct((NC, N, D), _f32),
        mesh=_sc_mesh(),
        scratch_types=[
            pltpu.VMEM((CHUNK,), jnp.int32),
            pltpu.VMEM((CHUNK,), jnp.int32),
            pltpu.VMEM((CHUNK, D), _f32),
            pltpu.VMEM_SHARED((N, D), _f32),
            pltpu.SemaphoreType.DMA,
            pltpu.SemaphoreType.DMA,
        ],
    )(msg, src1d, dst1d, zeros2d)


RB = 1000
NG = N // RB


def _t1_body(deg_ref, x_ref, w1_ref, m1_ref, dis_ref):
    deg = (jnp.sum(deg_ref[0, 0], axis=1)
           + jnp.sum(deg_ref[1, 0], axis=1)) * (1.0 / DW) + 1.0
    dis = lax.rsqrt(deg)
    m = jnp.dot(x_ref[...], w1_ref[...], preferred_element_type=_f32)
    m1_ref[...] = m * dis[:, None]
    dis_ref[0, 0, :] = dis


def _tc1(deg4d, x, W1):
    return pl.pallas_call(
        _t1_body,
        grid=(NG,),
        in_specs=[
            pl.BlockSpec((2, 1, RB, DW), lambda i: (0, i, 0, 0)),
            pl.BlockSpec((RB, DIN), lambda i: (i, 0)),
            pl.BlockSpec((DIN, DGH), lambda i: (0, 0)),
        ],
        out_specs=[
            pl.BlockSpec((RB, DGH), lambda i: (i, 0)),
            pl.BlockSpec((1, 1, RB), lambda i: (i, 0, 0)),
        ],
        out_shape=[
            jax.ShapeDtypeStruct((N, DGH), _f32),
            jax.ShapeDtypeStruct((NG, 1, RB), _f32),
        ],
    )(deg4d, x, W1)


def _t2_body(agg_ref, m1_ref, dis_ref, w2_ref, m2_ref):
    dis = dis_ref[0, 0, :]
    a = agg_ref[0] + agg_ref[1] + m1_ref[...]
    out1 = jnp.maximum(a * dis[:, None], 0.0)
    m2 = jnp.dot(out1, w2_ref[...], preferred_element_type=_f32) \
        * dis[:, None]
    m2_ref[...] = jnp.concatenate([m2, jnp.zeros((RB, DGH - DG), _f32)],
                                  axis=1)


def _tc2(agg1, m1, dis3d, W2):
    return pl.pallas_call(
        _t2_body,
        grid=(NG,),
        in_specs=[
            pl.BlockSpec((2, RB, DGH), lambda i: (0, i, 0)),
            pl.BlockSpec((RB, DGH), lambda i: (i, 0)),
            pl.BlockSpec((1, 1, RB), lambda i: (i, 0, 0)),
            pl.BlockSpec((DGH, DG), lambda i: (0, 0)),
        ],
        out_specs=pl.BlockSpec((RB, DGH), lambda i: (i, 0)),
        out_shape=jax.ShapeDtypeStruct((N, DGH), _f32),
    )(agg1, m1, dis3d, W2)


def _t3_body(agg_ref, m2_ref, dis_ref, desc_ref, logscale_ref, wm, out_ref,
             beta_ref, alpha_ref):
    eps = 1e-5
    Hf = (agg_ref[0] + agg_ref[1] + m2_ref[...])[:, :DG]
    dis2 = dis_ref[...]
    H3 = Hf.reshape(B, NPG, DG) * dis2[:, :, None]
    hg = jnp.mean(H3, axis=1)
    Hs = H3.reshape(N, DG)

    def ln32(t, g, b):
        mu = jnp.dot(t, wm['Gavg'], preferred_element_type=_f32)
        xm = t - mu
        var = jnp.dot(xm * xm, wm['Gavg'], preferred_element_type=_f32)
        return xm * lax.rsqrt(var + eps) * g + b

    tokpre = jnp.dot(desc_ref[...], wm['TokW'], preferred_element_type=_f32) \
        + wm['tokb']
    tok = jnp.maximum(ln32(tokpre, wm['tokg'], wm['tokB']), 0.0)

    kk = jnp.dot(Hs, wm['WkT'], preferred_element_type=_f32)
    v = jnp.dot(Hs, wm['WvT'], preferred_element_type=_f32)
    qf = jnp.dot(tok, wm['WqT'], preferred_element_type=_f32)
    qexp = jnp.broadcast_to(qf[:, None, :], (B, NPG, K * RANK)) \
        .reshape(N, K * RANK)
    scale = jnp.maximum(jnp.exp(logscale_ref[...]), 0.1)
    scores = jnp.dot(kk * qexp, wm['Gsum16'], preferred_element_type=_f32)
    scores3 = scores.reshape(B, NPG, K) / scale[0][None, None, :]
    mx = jnp.max(scores3, axis=1, keepdims=True)
    ex = jnp.exp(scores3 - mx)
    den = jnp.sum(ex, axis=1, keepdims=True)
    alpha3 = ex / den
    aexp = jnp.dot(alpha3.reshape(N, K), wm['Gexp32'],
                   preferred_element_type=_f32)
    ctx = jnp.sum((aexp * v).reshape(B, NPG, K * DH), axis=1)

    fused = (jnp.dot(ctx, wm['P0'], preferred_element_type=_f32)
             + jnp.dot(tok, wm['P1'], preferred_element_type=_f32)
             + jnp.dot(ctx * tok, wm['P2'], preferred_element_type=_f32)
             + jnp.dot(ctx - tok, wm['P3'], preferred_element_type=_f32))
    f1 = jnp.dot(fused, wm['FW1'], preferred_element_type=_f32) + wm['fb1']
    f1 = jnp.maximum(ln32(f1, wm['fg'], wm['fB']), 0.0)
    reps = jnp.dot(f1, wm['FW2'], preferred_element_type=_f32) + wm['fb2']

    gp = jnp.dot(hg, wm['agWT'], preferred_element_type=_f32) + wm['agb']
    fp = jnp.dot(reps, wm['AfT'], preferred_element_type=_f32) + wm['afb']
    gpexp = jnp.dot(gp, wm['Gtile32'], preferred_element_type=_f32)
    t = jnp.tanh(fp + gpexp)
    sc = jnp.dot(t, wm['AsT'], preferred_element_type=_f32) + wm['asb']
    mxs = jnp.max(sc, axis=1, keepdims=True)
    exs = jnp.exp(sc - mxs)
    beta = exs / jnp.sum(exs, axis=1, keepdims=True)
    bexp = jnp.dot(beta, wm['Gexp32'], preferred_element_type=_f32)
    h_fam = jnp.dot(bexp * reps, wm['Gfold'], preferred_element_type=_f32)

    z = (jnp.dot(hg, wm['Pz1'], preferred_element_type=_f32)
         + jnp.dot(h_fam, wm['Pz2'], preferred_element_type=_f32))
    z = jnp.maximum(jnp.dot(z, wm['hW1T'], preferred_element_type=_f32)
                    + wm['hb1'], 0.0)
    z = jnp.maximum(jnp.dot(z, wm['hW2T'], preferred_element_type=_f32)
                    + wm['hb2'], 0.0)
    out_ref[...] = jnp.dot(z, wm['hW3T'], preferred_element_type=_f32) \
        + wm['hb3']
    beta_ref[...] = beta
    alpha_ref[...] = jnp.swapaxes(alpha3, 1, 2)


def _tc3(agg2, m2, dis2d, desc, logscale2d, wmats):
    names = sorted(wmats.keys())
    vals = [wmats[k] for k in names]

    def body(agg_ref, m2_ref, dis_ref, desc_ref, ls_ref, *rest):
        wrefs = {k: r[...] for k, r in zip(names, rest[:len(names)])}
        out_ref, beta_ref, alpha_ref = rest[len(names):]
        _t3_body(agg_ref, m2_ref, dis_ref, desc_ref, ls_ref, wrefs,
                 out_ref, beta_ref, alpha_ref)

    return pl.pallas_call(
        body,
        out_shape=[
            jax.ShapeDtypeStruct((B, 1), _f32),
            jax.ShapeDtypeStruct((B, K), _f32),
            jax.ShapeDtypeStruct((B, K, NPG), _f32),
        ],
    )(agg2, m2, dis2d, desc, logscale2d, *vals)


def _prep_weights(p):
    bd = jax.scipy.linalg.block_diag
    i128 = jnp.arange(128)
    wm = {}
    wm['TokW'] = bd(*[p['tokW'][k].T for k in range(K)])
    wm['tokb'] = p['tokb'].reshape(-1)
    wm['tokg'] = p['tokg'].reshape(-1)
    wm['tokB'] = p['tokB'].reshape(-1)
    wm['Gavg'] = ((i128[:, None] // DH == i128[None, :] // DH)
                  .astype(_f32) / DH)
    wm['WkT'] = p['Wk'].transpose(2, 0, 1).reshape(DG, K * RANK)
    wm['WvT'] = p['Wv'].transpose(2, 0, 1).reshape(DG, K * DH)
    wm['WqT'] = bd(*[p['Wq'][k].T for k in range(K)])
    wm['Gsum16'] = (jnp.arange(K * RANK)[:, None] // RANK
                    == jnp.arange(K)[None, :]).astype(_f32)
    wm['Gexp32'] = (jnp.arange(K)[:, None]
                    == i128[None, :] // DH).astype(_f32)
    for pp in range(4):
        wm[f'P{pp}'] = (jnp.arange(K * 4 * DH)[None, :]
                        == (i128[:, None] // DH) * (4 * DH)
                        + pp * DH + i128[:, None] % DH).astype(_f32)
    wm['FW1'] = bd(*[p['fW1'][k].T for k in range(K)])
    wm['fb1'] = p['fb1'].reshape(-1)
    wm['fg'] = p['fg'].reshape(-1)
    wm['fB'] = p['fB'].reshape(-1)
    wm['FW2'] = bd(*[p['fW2'][k].T for k in range(K)])
    wm['fb2'] = p['fb2'].reshape(-1)
    wm['agWT'] = p['agW'].T
    wm['agb'] = p['agb']
    wm['AfT'] = bd(*[p['afW'].T] * K)
    wm['afb'] = jnp.tile(p['afb'], K)
    wm['Gtile32'] = (jnp.arange(DH)[:, None]
                     == i128[None, :] % DH).astype(_f32)
    wm['AsT'] = bd(*[p['asW'].T] * K)
    wm['asb'] = p['asb']
    wm['Gfold'] = (i128[:, None] % DH
                   == jnp.arange(DH)[None, :]).astype(_f32)
    wm['Pz1'] = jnp.concatenate([jnp.eye(DG, dtype=_f32),
                                 jnp.zeros((DG, DH), _f32)], axis=1)
    wm['Pz2'] = jnp.concatenate([jnp.zeros((DH, DG), _f32),
                                 jnp.eye(DH, dtype=_f32)], axis=1)
    wm['hW1T'] = p['hW1'].T
    wm['hb1'] = p['hb1']
    wm['hW2T'] = p['hW2'].T
    wm['hb2'] = p['hb2']
    wm['hW3T'] = p['hW3'].T
    wm['hb3'] = p['hb3']
    return wm


def kernel(x, edge_index, desc, params):
    p = params
    src1d = edge_index[0].astype(jnp.int32)
    dst1d = edge_index[1].astype(jnp.int32)
    zerosD = jnp.zeros((N, DW), _f32)
    onesD = jnp.ones((CHUNK, DW), _f32)
    zerosA = jnp.zeros((N, DGH), _f32)

    deg_parts = _sc_degree(dst1d, zerosD, onesD)
    deg4d = deg_parts.reshape(2, NG, RB, DW)

    m1, dis3d = _tc1(deg4d, x, p['W1'])
    agg1 = _sc_scatter(m1, src1d, dst1d, zerosA, DGH)
    m2 = _tc2(agg1, m1, dis3d, p['W2'])
    agg2 = _sc_scatter(m2, src1d, dst1d, zerosA, DGH)

    dis2d = dis3d.reshape(B, NPG)
    wm = _prep_weights(p)
    out, beta, alpha = _tc3(agg2, m2, dis2d, desc,
                            p['logscale'].reshape(1, K), wm)
    return out, beta, alpha


# speedup vs baseline: 7.4967x; 1.4973x over previous
import functools

import jax
import jax.numpy as jnp
from jax import lax
from jax.experimental import pallas as pl
from jax.experimental.pallas import tpu as pltpu
from jax.experimental.pallas import tpu_sc as plsc

N = 10000; E = 320000; B = 100; NPG = 100
DIN = 128; DGH = 128; DG = 64; DH = 32; RANK = 16; K = 4; FL = 49; DESC = 196

NC = 2
NS = 16
CHUNK = 128
NCHUNK = E // CHUNK
CPW = -(-NCHUNK // (NC * NS))
RPS = 640
RPS_LAST = N - RPS * (NS - 1)

_f32 = jnp.float32


def _sc_mesh():
    return plsc.VectorSubcoreMesh(core_axis_name="c", subcore_axis_name="s")


DW = 128


def _deg_body(dst_hbm, zeros_hbm, ones_hbm, out_hbm, dstv, onesv, acc_sh,
              gsem):
    c = lax.axis_index("c")
    s = lax.axis_index("s")
    w = s * NC + c

    pltpu.sync_copy(ones_hbm, onesv)

    @pl.when(s < NS - 1)
    def _():
        pltpu.sync_copy(zeros_hbm.at[pl.ds(s * RPS, RPS)],
                        acc_sh.at[pl.ds(s * RPS, RPS)])

    @pl.when(s == NS - 1)
    def _():
        pltpu.sync_copy(zeros_hbm.at[pl.ds((NS - 1) * RPS, RPS_LAST)],
                        acc_sh.at[pl.ds((NS - 1) * RPS, RPS_LAST)])

    plsc.subcore_barrier()

    def step(j, carry):
        ci = j * (NC * NS) + w

        @pl.when(ci < NCHUNK)
        def _():
            pltpu.async_copy(dst_hbm.at[pl.ds(ci * CHUNK, CHUNK)], dstv,
                             gsem).wait()
            pltpu.sync_copy(onesv, acc_sh.at[dstv], add=True)

        return carry

    lax.fori_loop(0, CPW, step, 0)
    plsc.subcore_barrier()

    @pl.when(s < NS - 1)
    def _():
        pltpu.sync_copy(acc_sh.at[pl.ds(s * RPS, RPS)],
                        out_hbm.at[c, pl.ds(s * RPS, RPS)])

    @pl.when(s == NS - 1)
    def _():
        pltpu.sync_copy(acc_sh.at[pl.ds((NS - 1) * RPS, RPS_LAST)],
                        out_hbm.at[c, pl.ds((NS - 1) * RPS, RPS_LAST)])


def _sc_degree(dst1d, zeros2d, ones2d):
    return pl.kernel(
        _deg_body,
        out_type=jax.ShapeDtypeStruct((NC, N, DW), _f32),
        mesh=_sc_mesh(),
        scratch_types=[
            pltpu.VMEM((CHUNK,), jnp.int32),
            pltpu.VMEM((CHUNK, DW), _f32),
            pltpu.VMEM_SHARED((N, DW), _f32),
            pltpu.SemaphoreType.DMA,
        ],
    )(dst1d, zeros2d, ones2d)


NFULL = 76
NTAIL = 3


def _scatter_body(D, msg_hbm, src_hbm, dst_hbm, zeros_hbm, out_hbm,
                  sv0, sv1, sv2, sv3, dv0, dv1, dv2, dv3, rows0, rows1,
                  acc_sh, gs0, gs1, is0, is1, is2, is3, ss0, ss1):
    srcv = [sv0, sv1, sv2, sv3]
    dstv = [dv0, dv1, dv2, dv3]
    rows = [rows0, rows1]
    gsems = [gs0, gs1]
    isems = [is0, is1, is2, is3]
    ssems = [ss0, ss1]
    c = lax.axis_index("c")
    s = lax.axis_index("s")
    w = s * NC + c
    NW = NC * NS

    @pl.when(s < NS - 1)
    def _():
        pltpu.sync_copy(zeros_hbm.at[pl.ds(s * RPS, RPS)],
                        acc_sh.at[pl.ds(s * RPS, RPS)])

    @pl.when(s == NS - 1)
    def _():
        pltpu.sync_copy(zeros_hbm.at[pl.ds((NS - 1) * RPS, RPS_LAST)],
                        acc_sh.at[pl.ds((NS - 1) * RPS, RPS_LAST)])

    plsc.subcore_barrier()

    def idx_start(j, b4):
        ci = j * NW + w
        c1 = pltpu.async_copy(src_hbm.at[pl.ds(ci * CHUNK, CHUNK)],
                              srcv[b4], isems[b4])
        c2 = pltpu.async_copy(dst_hbm.at[pl.ds(ci * CHUNK, CHUNK)],
                              dstv[b4], isems[b4])
        return c1, c2

    def idx_wait(b4):
        pltpu.make_async_copy(src_hbm.at[pl.ds(0, CHUNK)], srcv[b4],
                              isems[b4]).wait()
        pltpu.make_async_copy(dst_hbm.at[pl.ds(0, CHUNK)], dstv[b4],
                              isems[b4]).wait()

    idx_start(0, 0)
    idx_start(1, 1)

    def steady(jo, carry):
        for b in range(4):
            j = jo * 4 + b
            b2 = b % 2

            @pl.when(j >= 2)
            def _():
                pltpu.make_async_copy(rows[b2], acc_sh.at[dstv[b2]],
                                      ssems[b2]).wait()

            @pl.when(j + 2 < NFULL)
            def _():
                idx_start(j + 2, (b + 2) % 4)

            idx_wait(b)
            pltpu.async_copy(msg_hbm.at[srcv[b]], rows[b2],
                             gsems[b2])

            @pl.when(j >= 1)
            def _():
                pb2 = 1 - b2
                pb4 = (b - 1) % 4
                pltpu.make_async_copy(msg_hbm.at[srcv[pb4]],
                                      rows[pb2], gsems[pb2]).wait()
                pltpu.async_copy(rows[pb2], acc_sh.at[dstv[pb4]],
                                 ssems[pb2], add=True)

        return carry

    lax.fori_loop(0, NFULL // 4, steady, 0)

    lb2 = (NFULL - 1) % 2
    lb4 = (NFULL - 1) % 4
    pltpu.make_async_copy(msg_hbm.at[srcv[lb4]], rows[lb2],
                          gsems[lb2]).wait()
    pltpu.async_copy(rows[lb2], acc_sh.at[dstv[lb4]], ssems[lb2],
                     add=True)
    pltpu.make_async_copy(rows[0], acc_sh.at[dstv[0]], ssems[0]).wait()
    pltpu.make_async_copy(rows[1], acc_sh.at[dstv[1]], ssems[1]).wait()

    def tail(t, carry):
        ci = (NFULL + t) * NW + w

        @pl.when(ci < NCHUNK)
        def _():
            c1 = pltpu.async_copy(src_hbm.at[pl.ds(ci * CHUNK, CHUNK)],
                                  srcv[0], isems[0])
            c2 = pltpu.async_copy(dst_hbm.at[pl.ds(ci * CHUNK, CHUNK)],
                                  dstv[0], isems[0])
            c1.wait()
            c2.wait()
            pltpu.async_copy(msg_hbm.at[srcv[0]], rows[0],
                             gsems[0]).wait()
            pltpu.async_copy(rows[0], acc_sh.at[dstv[0]], ssems[0],
                             add=True).wait()

        return carry

    lax.fori_loop(0, NTAIL, tail, 0)
    plsc.subcore_barrier()

    @pl.when(s < NS - 1)
    def _():
        pltpu.sync_copy(acc_sh.at[pl.ds(s * RPS, RPS)],
                        out_hbm.at[c, pl.ds(s * RPS, RPS)])

    @pl.when(s == NS - 1)
    def _():
        pltpu.sync_copy(acc_sh.at[pl.ds((NS - 1) * RPS, RPS_LAST)],
                        out_hbm.at[c, pl.ds((NS - 1) * RPS, RPS_LAST)])


def _sc_scatter(msg, src1d, dst1d, zeros2d, D):
    body = functools.partial(_scatter_body, D)
    return pl.kernel(
        body,
        out_type=jax.ShapeDtypeStruct((NC, N, D), _f32),
        mesh=_sc_mesh(),
        scratch_types=[
            pltpu.VMEM((CHUNK,), jnp.int32),
            pltpu.VMEM((CHUNK,), jnp.int32),
            pltpu.VMEM((CHUNK,), jnp.int32),
            pltpu.VMEM((CHUNK,), jnp.int32),
            pltpu.VMEM((CHUNK,), jnp.int32),
            pltpu.VMEM((CHUNK,), jnp.int32),
            pltpu.VMEM((CHUNK,), jnp.int32),
            pltpu.VMEM((CHUNK,), jnp.int32),
            pltpu.VMEM((CHUNK, D), _f32),
            pltpu.VMEM((CHUNK, D), _f32),
            pltpu.VMEM_SHARED((N, D), _f32),
            pltpu.SemaphoreType.DMA,
            pltpu.SemaphoreType.DMA,
            pltpu.SemaphoreType.DMA,
            pltpu.SemaphoreType.DMA,
            pltpu.SemaphoreType.DMA,
            pltpu.SemaphoreType.DMA,
            pltpu.SemaphoreType.DMA,
            pltpu.SemaphoreType.DMA,
        ],
    )(msg, src1d, dst1d, zeros2d)


RB = 1000
NG = N // RB


def _t1_body(deg_ref, x_ref, w1_ref, m1_ref, dis_ref):
    deg = (jnp.sum(deg_ref[0, 0], axis=1)
           + jnp.sum(deg_ref[1, 0], axis=1)) * (1.0 / DW) + 1.0
    dis = lax.rsqrt(deg)
    m = jnp.dot(x_ref[...], w1_ref[...], preferred_element_type=_f32)
    m1_ref[...] = m * dis[:, None]
    dis_ref[0, 0, :] = dis


def _tc1(deg4d, x, W1):
    return pl.pallas_call(
        _t1_body,
        grid=(NG,),
        in_specs=[
            pl.BlockSpec((2, 1, RB, DW), lambda i: (0, i, 0, 0)),
            pl.BlockSpec((RB, DIN), lambda i: (i, 0)),
            pl.BlockSpec((DIN, DGH), lambda i: (0, 0)),
        ],
        out_specs=[
            pl.BlockSpec((RB, DGH), lambda i: (i, 0)),
            pl.BlockSpec((1, 1, RB), lambda i: (i, 0, 0)),
        ],
        out_shape=[
            jax.ShapeDtypeStruct((N, DGH), _f32),
            jax.ShapeDtypeStruct((NG, 1, RB), _f32),
        ],
    )(deg4d, x, W1)


def _t2_body(agg_ref, m1_ref, dis_ref, w2_ref, m2_ref):
    dis = dis_ref[0, 0, :]
    a = agg_ref[0] + agg_ref[1] + m1_ref[...]
    out1 = jnp.maximum(a * dis[:, None], 0.0)
    m2 = jnp.dot(out1, w2_ref[...], preferred_element_type=_f32) \
        * dis[:, None]
    m2_ref[...] = jnp.concatenate([m2, jnp.zeros((RB, DGH - DG), _f32)],
                                  axis=1)


def _tc2(agg1, m1, dis3d, W2):
    return pl.pallas_call(
        _t2_body,
        grid=(NG,),
        in_specs=[
            pl.BlockSpec((2, RB, DGH), lambda i: (0, i, 0)),
            pl.BlockSpec((RB, DGH), lambda i: (i, 0)),
            pl.BlockSpec((1, 1, RB), lambda i: (i, 0, 0)),
            pl.BlockSpec((DGH, DG), lambda i: (0, 0)),
        ],
        out_specs=pl.BlockSpec((RB, DGH), lambda i: (i, 0)),
        out_shape=jax.ShapeDtypeStruct((N, DGH), _f32),
    )(agg1, m1, dis3d, W2)


def _t3_body(agg_ref, m2_ref, dis_ref, desc_ref, logscale_ref, wm, out_ref,
             beta_ref, alpha_ref):
    eps = 1e-5
    Hf = (agg_ref[0] + agg_ref[1] + m2_ref[...])[:, :DG]
    dis2 = dis_ref[...]
    H3 = Hf.reshape(B, NPG, DG) * dis2[:, :, None]
    hg = jnp.mean(H3, axis=1)
    Hs = H3.reshape(N, DG)

    def ln32(t, g, b):
        mu = jnp.dot(t, wm['Gavg'], preferred_element_type=_f32)
        xm = t - mu
        var = jnp.dot(xm * xm, wm['Gavg'], preferred_element_type=_f32)
        return xm * lax.rsqrt(var + eps) * g + b

    tokpre = jnp.dot(desc_ref[...], wm['TokW'], preferred_element_type=_f32) \
        + wm['tokb']
    tok = jnp.maximum(ln32(tokpre, wm['tokg'], wm['tokB']), 0.0)

    kk = jnp.dot(Hs, wm['WkT'], preferred_element_type=_f32)
    v = jnp.dot(Hs, wm['WvT'], preferred_element_type=_f32)
    qf = jnp.dot(tok, wm['WqT'], preferred_element_type=_f32)
    qexp = jnp.broadcast_to(qf[:, None, :], (B, NPG, K * RANK)) \
        .reshape(N, K * RANK)
    scale = jnp.maximum(jnp.exp(logscale_ref[...]), 0.1)
    scores = jnp.dot(kk * qexp, wm['Gsum16'], preferred_element_type=_f32)
    scores3 = scores.reshape(B, NPG, K) / scale[0][None, None, :]
    mx = jnp.max(scores3, axis=1, keepdims=True)
    ex = jnp.exp(scores3 - mx)
    den = jnp.sum(ex, axis=1, keepdims=True)
    alpha3 = ex / den
    aexp = jnp.dot(alpha3.reshape(N, K), wm['Gexp32'],
                   preferred_element_type=_f32)
    ctx = jnp.sum((aexp * v).reshape(B, NPG, K * DH), axis=1)

    fused = (jnp.dot(ctx, wm['P0'], preferred_element_type=_f32)
             + jnp.dot(tok, wm['P1'], preferred_element_type=_f32)
             + jnp.dot(ctx * tok, wm['P2'], preferred_element_type=_f32)
             + jnp.dot(ctx - tok, wm['P3'], preferred_element_type=_f32))
    f1 = jnp.dot(fused, wm['FW1'], preferred_element_type=_f32) + wm['fb1']
    f1 = jnp.maximum(ln32(f1, wm['fg'], wm['fB']), 0.0)
    reps = jnp.dot(f1, wm['FW2'], preferred_element_type=_f32) + wm['fb2']

    gp = jnp.dot(hg, wm['agWT'], preferred_element_type=_f32) + wm['agb']
    fp = jnp.dot(reps, wm['AfT'], preferred_element_type=_f32) + wm['afb']
    gpexp = jnp.dot(gp, wm['Gtile32'], preferred_element_type=_f32)
    t = jnp.tanh(fp + gpexp)
    sc = jnp.dot(t, wm['AsT'], preferred_element_type=_f32) + wm['asb']
    mxs = jnp.max(sc, axis=1, keepdims=True)
    exs = jnp.exp(sc - mxs)
    beta = exs / jnp.sum(exs, axis=1, keepdims=True)
    bexp = jnp.dot(beta, wm['Gexp32'], preferred_element_type=_f32)
    h_fam = jnp.dot(bexp * reps, wm['Gfold'], preferred_element_type=_f32)

    z = (jnp.dot(hg, wm['Pz1'], preferred_element_type=_f32)
         + jnp.dot(h_fam, wm['Pz2'], preferred_element_type=_f32))
    z = jnp.maximum(jnp.dot(z, wm['hW1T'], preferred_element_type=_f32)
                    + wm['hb1'], 0.0)
    z = jnp.maximum(jnp.dot(z, wm['hW2T'], preferred_element_type=_f32)
                    + wm['hb2'], 0.0)
    out_ref[...] = jnp.dot(z, wm['hW3T'], preferred_element_type=_f32) \
        + wm['hb3']
    beta_ref[...] = beta
    alpha_ref[...] = jnp.swapaxes(alpha3, 1, 2)


def _tc3(agg2, m2, dis2d, desc, logscale2d, wmats):
    names = sorted(wmats.keys())
    vals = [wmats[k] for k in names]

    def body(agg_ref, m2_ref, dis_ref, desc_ref, ls_ref, *rest):
        wrefs = {k: r[...] for k, r in zip(names, rest[:len(names)])}
        out_ref, beta_ref, alpha_ref = rest[len(names):]
        _t3_body(agg_ref, m2_ref, dis_ref, desc_ref, ls_ref, wrefs,
                 out_ref, beta_ref, alpha_ref)

    return pl.pallas_call(
        body,
        out_shape=[
            jax.ShapeDtypeStruct((B, 1), _f32),
            jax.ShapeDtypeStruct((B, K), _f32),
            jax.ShapeDtypeStruct((B, K, NPG), _f32),
        ],
    )(agg2, m2, dis2d, desc, logscale2d, *vals)


def _prep_weights(p):
    bd = jax.scipy.linalg.block_diag
    i128 = jnp.arange(128)
    wm = {}
    wm['TokW'] = bd(*[p['tokW'][k].T for k in range(K)])
    wm['tokb'] = p['tokb'].reshape(-1)
    wm['tokg'] = p['tokg'].reshape(-1)
    wm['tokB'] = p['tokB'].reshape(-1)
    wm['Gavg'] = ((i128[:, None] // DH == i128[None, :] // DH)
                  .astype(_f32) / DH)
    wm['WkT'] = p['Wk'].transpose(2, 0, 1).reshape(DG, K * RANK)
    wm['WvT'] = p['Wv'].transpose(2, 0, 1).reshape(DG, K * DH)
    wm['WqT'] = bd(*[p['Wq'][k].T for k in range(K)])
    wm['Gsum16'] = (jnp.arange(K * RANK)[:, None] // RANK
                    == jnp.arange(K)[None, :]).astype(_f32)
    wm['Gexp32'] = (jnp.arange(K)[:, None]
                    == i128[None, :] // DH).astype(_f32)
    for pp in range(4):
        wm[f'P{pp}'] = (jnp.arange(K * 4 * DH)[None, :]
                        == (i128[:, None] // DH) * (4 * DH)
                        + pp * DH + i128[:, None] % DH).astype(_f32)
    wm['FW1'] = bd(*[p['fW1'][k].T for k in range(K)])
    wm['fb1'] = p['fb1'].reshape(-1)
    wm['fg'] = p['fg'].reshape(-1)
    wm['fB'] = p['fB'].reshape(-1)
    wm['FW2'] = bd(*[p['fW2'][k].T for k in range(K)])
    wm['fb2'] = p['fb2'].reshape(-1)
    wm['agWT'] = p['agW'].T
    wm['agb'] = p['agb']
    wm['AfT'] = bd(*[p['afW'].T] * K)
    wm['afb'] = jnp.tile(p['afb'], K)
    wm['Gtile32'] = (jnp.arange(DH)[:, None]
                     == i128[None, :] % DH).astype(_f32)
    wm['AsT'] = bd(*[p['asW'].T] * K)
    wm['asb'] = p['asb']
    wm['Gfold'] = (i128[:, None] % DH
                   == jnp.arange(DH)[None, :]).astype(_f32)
    wm['Pz1'] = jnp.concatenate([jnp.eye(DG, dtype=_f32),
                                 jnp.zeros((DG, DH), _f32)], axis=1)
    wm['Pz2'] = jnp.concatenate([jnp.zeros((DH, DG), _f32),
                                 jnp.eye(DH, dtype=_f32)], axis=1)
    wm['hW1T'] = p['hW1'].T
    wm['hb1'] = p['hb1']
    wm['hW2T'] = p['hW2'].T
    wm['hb2'] = p['hb2']
    wm['hW3T'] = p['hW3'].T
    wm['hb3'] = p['hb3']
    return wm


def kernel(x, edge_index, desc, params):
    p = params
    src1d = edge_index[0].astype(jnp.int32)
    dst1d = edge_index[1].astype(jnp.int32)
    zerosD = jnp.zeros((N, DW), _f32)
    onesD = jnp.ones((CHUNK, DW), _f32)
    zerosA = jnp.zeros((N, DGH), _f32)

    deg_parts = _sc_degree(dst1d, zerosD, onesD)
    deg4d = deg_parts.reshape(2, NG, RB, DW)

    m1, dis3d = _tc1(deg4d, x, p['W1'])
    agg1 = _sc_scatter(m1, src1d, dst1d, zerosA, DGH)
    m2 = _tc2(agg1, m1, dis3d, p['W2'])
    agg2 = _sc_scatter(m2, src1d, dst1d, zerosA, DGH)

    dis2d = dis3d.reshape(B, NPG)
    wm = _prep_weights(p)
    out, beta, alpha = _tc3(agg2, m2, dis2d, desc,
                            p['logscale'].reshape(1, K), wm)
    return out, beta, alpha


# speedup vs baseline: 8.1273x; 1.0841x over previous
import functools

import jax
import jax.numpy as jnp
from jax import lax
from jax.experimental import pallas as pl
from jax.experimental.pallas import tpu as pltpu
from jax.experimental.pallas import tpu_sc as plsc

N = 10000; E = 320000; B = 100; NPG = 100
DIN = 128; DGH = 128; DG = 64; DH = 32; RANK = 16; K = 4; FL = 49; DESC = 196

NC = 2
NS = 16
CHUNK = 128
NCHUNK = E // CHUNK
CPW = -(-NCHUNK // (NC * NS))
RPS = 640
RPS_LAST = N - RPS * (NS - 1)

_f32 = jnp.float32


def _sc_mesh():
    return plsc.VectorSubcoreMesh(core_axis_name="c", subcore_axis_name="s")


DW = 128


def _deg_body(dst_hbm, zeros_hbm, ones_hbm, out_hbm, dv0, dv1, dv2, dv3,
              onesv, acc_sh, is0, is1, is2, is3, ss0, ss1):
    dstv = [dv0, dv1, dv2, dv3]
    isems = [is0, is1, is2, is3]
    ssems = [ss0, ss1]
    c = lax.axis_index("c")
    s = lax.axis_index("s")
    w = s * NC + c
    NW = NC * NS

    pltpu.sync_copy(ones_hbm, onesv)

    @pl.when(s < NS - 1)
    def _():
        pltpu.sync_copy(zeros_hbm.at[pl.ds(s * RPS, RPS)],
                        acc_sh.at[pl.ds(s * RPS, RPS)])

    @pl.when(s == NS - 1)
    def _():
        pltpu.sync_copy(zeros_hbm.at[pl.ds((NS - 1) * RPS, RPS_LAST)],
                        acc_sh.at[pl.ds((NS - 1) * RPS, RPS_LAST)])

    plsc.subcore_barrier()

    def idx_start(j, b4):
        ci = j * NW + w
        pltpu.async_copy(dst_hbm.at[pl.ds(ci * CHUNK, CHUNK)], dstv[b4],
                         isems[b4])

    def idx_wait(b4):
        pltpu.make_async_copy(dst_hbm.at[pl.ds(0, CHUNK)], dstv[b4],
                              isems[b4]).wait()

    idx_start(0, 0)
    idx_start(1, 1)

    def steady(jo, carry):
        for b in range(4):
            j = jo * 4 + b
            b2 = b % 2

            @pl.when(j >= 2)
            def _():
                pltpu.make_async_copy(onesv, acc_sh.at[dstv[b2]],
                                      ssems[b2]).wait()

            @pl.when(j + 2 < NFULL)
            def _():
                idx_start(j + 2, (b + 2) % 4)

            idx_wait(b)
            pltpu.async_copy(onesv, acc_sh.at[dstv[b]], ssems[b2],
                             add=True)

        return carry

    lax.fori_loop(0, NFULL // 4, steady, 0)
    pltpu.make_async_copy(onesv, acc_sh.at[dstv[0]], ssems[0]).wait()
    pltpu.make_async_copy(onesv, acc_sh.at[dstv[1]], ssems[1]).wait()

    def tail(t, carry):
        ci = (NFULL + t) * NW + w

        @pl.when(ci < NCHUNK)
        def _():
            pltpu.async_copy(dst_hbm.at[pl.ds(ci * CHUNK, CHUNK)], dstv[0],
                             isems[0]).wait()
            pltpu.async_copy(onesv, acc_sh.at[dstv[0]], ssems[0],
                             add=True).wait()

        return carry

    lax.fori_loop(0, NTAIL, tail, 0)
    plsc.subcore_barrier()

    @pl.when(s < NS - 1)
    def _():
        pltpu.sync_copy(acc_sh.at[pl.ds(s * RPS, RPS)],
                        out_hbm.at[c, pl.ds(s * RPS, RPS)])

    @pl.when(s == NS - 1)
    def _():
        pltpu.sync_copy(acc_sh.at[pl.ds((NS - 1) * RPS, RPS_LAST)],
                        out_hbm.at[c, pl.ds((NS - 1) * RPS, RPS_LAST)])


def _sc_degree(dst1d, zeros2d, ones2d):
    return pl.kernel(
        _deg_body,
        out_type=jax.ShapeDtypeStruct((NC, N, DW), _f32),
        mesh=_sc_mesh(),
        scratch_types=[
            pltpu.VMEM((CHUNK,), jnp.int32),
            pltpu.VMEM((CHUNK,), jnp.int32),
            pltpu.VMEM((CHUNK,), jnp.int32),
            pltpu.VMEM((CHUNK,), jnp.int32),
            pltpu.VMEM((CHUNK, DW), _f32),
            pltpu.VMEM_SHARED((N, DW), _f32),
            pltpu.SemaphoreType.DMA,
            pltpu.SemaphoreType.DMA,
            pltpu.SemaphoreType.DMA,
            pltpu.SemaphoreType.DMA,
            pltpu.SemaphoreType.DMA,
            pltpu.SemaphoreType.DMA,
        ],
    )(dst1d, zeros2d, ones2d)


NFULL = 76
NTAIL = 3


def _scatter_body(D, msg_hbm, src_hbm, dst_hbm, zeros_hbm, out_hbm,
                  sv0, sv1, sv2, sv3, dv0, dv1, dv2, dv3, rows0, rows1,
                  acc_sh, gs0, gs1, is0, is1, is2, is3, ss0, ss1):
    srcv = [sv0, sv1, sv2, sv3]
    dstv = [dv0, dv1, dv2, dv3]
    rows = [rows0, rows1]
    gsems = [gs0, gs1]
    isems = [is0, is1, is2, is3]
    ssems = [ss0, ss1]
    c = lax.axis_index("c")
    s = lax.axis_index("s")
    w = s * NC + c
    NW = NC * NS

    @pl.when(s < NS - 1)
    def _():
        pltpu.sync_copy(zeros_hbm.at[pl.ds(s * RPS, RPS)],
                        acc_sh.at[pl.ds(s * RPS, RPS)])

    @pl.when(s == NS - 1)
    def _():
        pltpu.sync_copy(zeros_hbm.at[pl.ds((NS - 1) * RPS, RPS_LAST)],
                        acc_sh.at[pl.ds((NS - 1) * RPS, RPS_LAST)])

    plsc.subcore_barrier()

    def idx_start(j, b4):
        ci = j * NW + w
        c1 = pltpu.async_copy(src_hbm.at[pl.ds(ci * CHUNK, CHUNK)],
                              srcv[b4], isems[b4])
        c2 = pltpu.async_copy(dst_hbm.at[pl.ds(ci * CHUNK, CHUNK)],
                              dstv[b4], isems[b4])
        return c1, c2

    def idx_wait(b4):
        pltpu.make_async_copy(src_hbm.at[pl.ds(0, CHUNK)], srcv[b4],
                              isems[b4]).wait()
        pltpu.make_async_copy(dst_hbm.at[pl.ds(0, CHUNK)], dstv[b4],
                              isems[b4]).wait()

    idx_start(0, 0)
    idx_start(1, 1)

    def steady(jo, carry):
        for b in range(4):
            j = jo * 4 + b
            b2 = b % 2

            @pl.when(j >= 2)
            def _():
                pltpu.make_async_copy(rows[b2], acc_sh.at[dstv[b2]],
                                      ssems[b2]).wait()

            @pl.when(j + 2 < NFULL)
            def _():
                idx_start(j + 2, (b + 2) % 4)

            idx_wait(b)
            pltpu.async_copy(msg_hbm.at[srcv[b]], rows[b2],
                             gsems[b2])

            @pl.when(j >= 1)
            def _():
                pb2 = 1 - b2
                pb4 = (b - 1) % 4
                pltpu.make_async_copy(msg_hbm.at[srcv[pb4]],
                                      rows[pb2], gsems[pb2]).wait()
                pltpu.async_copy(rows[pb2], acc_sh.at[dstv[pb4]],
                                 ssems[pb2], add=True)

        return carry

    lax.fori_loop(0, NFULL // 4, steady, 0)

    lb2 = (NFULL - 1) % 2
    lb4 = (NFULL - 1) % 4
    pltpu.make_async_copy(msg_hbm.at[srcv[lb4]], rows[lb2],
                          gsems[lb2]).wait()
    pltpu.async_copy(rows[lb2], acc_sh.at[dstv[lb4]], ssems[lb2],
                     add=True)
    pltpu.make_async_copy(rows[0], acc_sh.at[dstv[0]], ssems[0]).wait()
    pltpu.make_async_copy(rows[1], acc_sh.at[dstv[1]], ssems[1]).wait()

    def tail(t, carry):
        ci = (NFULL + t) * NW + w

        @pl.when(ci < NCHUNK)
        def _():
            c1 = pltpu.async_copy(src_hbm.at[pl.ds(ci * CHUNK, CHUNK)],
                                  srcv[0], isems[0])
            c2 = pltpu.async_copy(dst_hbm.at[pl.ds(ci * CHUNK, CHUNK)],
                                  dstv[0], isems[0])
            c1.wait()
            c2.wait()
            pltpu.async_copy(msg_hbm.at[srcv[0]], rows[0],
                             gsems[0]).wait()
            pltpu.async_copy(rows[0], acc_sh.at[dstv[0]], ssems[0],
                             add=True).wait()

        return carry

    lax.fori_loop(0, NTAIL, tail, 0)
    plsc.subcore_barrier()

    @pl.when(s < NS - 1)
    def _():
        pltpu.sync_copy(acc_sh.at[pl.ds(s * RPS, RPS)],
                        out_hbm.at[c, pl.ds(s * RPS, RPS)])

    @pl.when(s == NS - 1)
    def _():
        pltpu.sync_copy(acc_sh.at[pl.ds((NS - 1) * RPS, RPS_LAST)],
                        out_hbm.at[c, pl.ds((NS - 1) * RPS, RPS_LAST)])


def _sc_scatter(msg, src1d, dst1d, zeros2d, D):
    body = functools.partial(_scatter_body, D)
    return pl.kernel(
        body,
        out_type=jax.ShapeDtypeStruct((NC, N, D), _f32),
        mesh=_sc_mesh(),
        scratch_types=[
            pltpu.VMEM((CHUNK,), jnp.int32),
            pltpu.VMEM((CHUNK,), jnp.int32),
            pltpu.VMEM((CHUNK,), jnp.int32),
            pltpu.VMEM((CHUNK,), jnp.int32),
            pltpu.VMEM((CHUNK,), jnp.int32),
            pltpu.VMEM((CHUNK,), jnp.int32),
            pltpu.VMEM((CHUNK,), jnp.int32),
            pltpu.VMEM((CHUNK,), jnp.int32),
            pltpu.VMEM((CHUNK, D), _f32),
            pltpu.VMEM((CHUNK, D), _f32),
            pltpu.VMEM_SHARED((N, D), _f32),
            pltpu.SemaphoreType.DMA,
            pltpu.SemaphoreType.DMA,
            pltpu.SemaphoreType.DMA,
            pltpu.SemaphoreType.DMA,
            pltpu.SemaphoreType.DMA,
            pltpu.SemaphoreType.DMA,
            pltpu.SemaphoreType.DMA,
            pltpu.SemaphoreType.DMA,
        ],
    )(msg, src1d, dst1d, zeros2d)


RB = 1000
NG = N // RB


def _t1_body(deg_ref, x_ref, w1_ref, m1_ref, dis_ref):
    deg = (jnp.sum(deg_ref[0, 0], axis=1)
           + jnp.sum(deg_ref[1, 0], axis=1)) * (1.0 / DW) + 1.0
    dis = lax.rsqrt(deg)
    m = jnp.dot(x_ref[...], w1_ref[...], preferred_element_type=_f32)
    m1_ref[...] = m * dis[:, None]
    dis_ref[0, 0, :] = dis


def _tc1(deg4d, x, W1):
    return pl.pallas_call(
        _t1_body,
        grid=(NG,),
        in_specs=[
            pl.BlockSpec((2, 1, RB, DW), lambda i: (0, i, 0, 0)),
            pl.BlockSpec((RB, DIN), lambda i: (i, 0)),
            pl.BlockSpec((DIN, DGH), lambda i: (0, 0)),
        ],
        out_specs=[
            pl.BlockSpec((RB, DGH), lambda i: (i, 0)),
            pl.BlockSpec((1, 1, RB), lambda i: (i, 0, 0)),
        ],
        out_shape=[
            jax.ShapeDtypeStruct((N, DGH), _f32),
            jax.ShapeDtypeStruct((NG, 1, RB), _f32),
        ],
    )(deg4d, x, W1)


def _t2_body(agg_ref, m1_ref, dis_ref, w2_ref, m2_ref):
    dis = dis_ref[0, 0, :]
    a = agg_ref[0] + agg_ref[1] + m1_ref[...]
    out1 = jnp.maximum(a * dis[:, None], 0.0)
    m2 = jnp.dot(out1, w2_ref[...], preferred_element_type=_f32) \
        * dis[:, None]
    m2_ref[...] = jnp.concatenate([m2, jnp.zeros((RB, DGH - DG), _f32)],
                                  axis=1)


def _tc2(agg1, m1, dis3d, W2):
    return pl.pallas_call(
        _t2_body,
        grid=(NG,),
        in_specs=[
            pl.BlockSpec((2, RB, DGH), lambda i: (0, i, 0)),
            pl.BlockSpec((RB, DGH), lambda i: (i, 0)),
            pl.BlockSpec((1, 1, RB), lambda i: (i, 0, 0)),
            pl.BlockSpec((DGH, DG), lambda i: (0, 0)),
        ],
        out_specs=pl.BlockSpec((RB, DGH), lambda i: (i, 0)),
        out_shape=jax.ShapeDtypeStruct((N, DGH), _f32),
    )(agg1, m1, dis3d, W2)


def _t3_body(agg_ref, m2_ref, dis_ref, desc_ref, logscale_ref, wm, out_ref,
             beta_ref, alpha_ref):
    eps = 1e-5
    Hf = (agg_ref[0] + agg_ref[1] + m2_ref[...])[:, :DG]
    dis2 = dis_ref[...]
    H3 = Hf.reshape(B, NPG, DG) * dis2[:, :, None]
    hg = jnp.mean(H3, axis=1)
    Hs = H3.reshape(N, DG)

    def ln32(t, g, b):
        mu = jnp.dot(t, wm['Gavg'], preferred_element_type=_f32)
        xm = t - mu
        var = jnp.dot(xm * xm, wm['Gavg'], preferred_element_type=_f32)
        return xm * lax.rsqrt(var + eps) * g + b

    tokpre = jnp.dot(desc_ref[...], wm['TokW'], preferred_element_type=_f32) \
        + wm['tokb']
    tok = jnp.maximum(ln32(tokpre, wm['tokg'], wm['tokB']), 0.0)

    kk = jnp.dot(Hs, wm['WkT'], preferred_element_type=_f32)
    v = jnp.dot(Hs, wm['WvT'], preferred_element_type=_f32)
    qf = jnp.dot(tok, wm['WqT'], preferred_element_type=_f32)
    qexp = jnp.broadcast_to(qf[:, None, :], (B, NPG, K * RANK)) \
        .reshape(N, K * RANK)
    scale = jnp.maximum(jnp.exp(logscale_ref[...]), 0.1)
    scores = jnp.dot(kk * qexp, wm['Gsum16'], preferred_element_type=_f32)
    scores3 = scores.reshape(B, NPG, K) / scale[0][None, None, :]
    mx = jnp.max(scores3, axis=1, keepdims=True)
    ex = jnp.exp(scores3 - mx)
    den = jnp.sum(ex, axis=1, keepdims=True)
    alpha3 = ex / den
    aexp = jnp.dot(alpha3.reshape(N, K), wm['Gexp32'],
                   preferred_element_type=_f32)
    ctx = jnp.sum((aexp * v).reshape(B, NPG, K * DH), axis=1)

    fused = (jnp.dot(ctx, wm['P0'], preferred_element_type=_f32)
             + jnp.dot(tok, wm['P1'], preferred_element_type=_f32)
             + jnp.dot(ctx * tok, wm['P2'], preferred_element_type=_f32)
             + jnp.dot(ctx - tok, wm['P3'], preferred_element_type=_f32))
    f1 = jnp.dot(fused, wm['FW1'], preferred_element_type=_f32) + wm['fb1']
    f1 = jnp.maximum(ln32(f1, wm['fg'], wm['fB']), 0.0)
    reps = jnp.dot(f1, wm['FW2'], preferred_element_type=_f32) + wm['fb2']

    gp = jnp.dot(hg, wm['agWT'], preferred_element_type=_f32) + wm['agb']
    fp = jnp.dot(reps, wm['AfT'], preferred_element_type=_f32) + wm['afb']
    gpexp = jnp.dot(gp, wm['Gtile32'], preferred_element_type=_f32)
    t = jnp.tanh(fp + gpexp)
    sc = jnp.dot(t, wm['AsT'], preferred_element_type=_f32) + wm['asb']
    mxs = jnp.max(sc, axis=1, keepdims=True)
    exs = jnp.exp(sc - mxs)
    beta = exs / jnp.sum(exs, axis=1, keepdims=True)
    bexp = jnp.dot(beta, wm['Gexp32'], preferred_element_type=_f32)
    h_fam = jnp.dot(bexp * reps, wm['Gfold'], preferred_element_type=_f32)

    z = (jnp.dot(hg, wm['Pz1'], preferred_element_type=_f32)
         + jnp.dot(h_fam, wm['Pz2'], preferred_element_type=_f32))
    z = jnp.maximum(jnp.dot(z, wm['hW1T'], preferred_element_type=_f32)
                    + wm['hb1'], 0.0)
    z = jnp.maximum(jnp.dot(z, wm['hW2T'], preferred_element_type=_f32)
                    + wm['hb2'], 0.0)
    out_ref[...] = jnp.dot(z, wm['hW3T'], preferred_element_type=_f32) \
        + wm['hb3']
    beta_ref[...] = beta
    alpha_ref[...] = jnp.swapaxes(alpha3, 1, 2)


def _tc3(agg2, m2, dis2d, desc, logscale2d, wmats):
    names = sorted(wmats.keys())
    vals = [wmats[k] for k in names]

    def body(agg_ref, m2_ref, dis_ref, desc_ref, ls_ref, *rest):
        wrefs = {k: r[...] for k, r in zip(names, rest[:len(names)])}
        out_ref, beta_ref, alpha_ref = rest[len(names):]
        _t3_body(agg_ref, m2_ref, dis_ref, desc_ref, ls_ref, wrefs,
                 out_ref, beta_ref, alpha_ref)

    return pl.pallas_call(
        body,
        out_shape=[
            jax.ShapeDtypeStruct((B, 1), _f32),
            jax.ShapeDtypeStruct((B, K), _f32),
            jax.ShapeDtypeStruct((B, K, NPG), _f32),
        ],
    )(agg2, m2, dis2d, desc, logscale2d, *vals)


def _prep_weights(p):
    bd = jax.scipy.linalg.block_diag
    i128 = jnp.arange(128)
    wm = {}
    wm['TokW'] = bd(*[p['tokW'][k].T for k in range(K)])
    wm['tokb'] = p['tokb'].reshape(-1)
    wm['tokg'] = p['tokg'].reshape(-1)
    wm['tokB'] = p['tokB'].reshape(-1)
    wm['Gavg'] = ((i128[:, None] // DH == i128[None, :] // DH)
                  .astype(_f32) / DH)
    wm['WkT'] = p['Wk'].transpose(2, 0, 1).reshape(DG, K * RANK)
    wm['WvT'] = p['Wv'].transpose(2, 0, 1).reshape(DG, K * DH)
    wm['WqT'] = bd(*[p['Wq'][k].T for k in range(K)])
    wm['Gsum16'] = (jnp.arange(K * RANK)[:, None] // RANK
                    == jnp.arange(K)[None, :]).astype(_f32)
    wm['Gexp32'] = (jnp.arange(K)[:, None]
                    == i128[None, :] // DH).astype(_f32)
    for pp in range(4):
        wm[f'P{pp}'] = (jnp.arange(K * 4 * DH)[None, :]
                        == (i128[:, None] // DH) * (4 * DH)
                        + pp * DH + i128[:, None] % DH).astype(_f32)
    wm['FW1'] = bd(*[p['fW1'][k].T for k in range(K)])
    wm['fb1'] = p['fb1'].reshape(-1)
    wm['fg'] = p['fg'].reshape(-1)
    wm['fB'] = p['fB'].reshape(-1)
    wm['FW2'] = bd(*[p['fW2'][k].T for k in range(K)])
    wm['fb2'] = p['fb2'].reshape(-1)
    wm['agWT'] = p['agW'].T
    wm['agb'] = p['agb']
    wm['AfT'] = bd(*[p['afW'].T] * K)
    wm['afb'] = jnp.tile(p['afb'], K)
    wm['Gtile32'] = (jnp.arange(DH)[:, None]
                     == i128[None, :] % DH).astype(_f32)
    wm['AsT'] = bd(*[p['asW'].T] * K)
    wm['asb'] = p['asb']
    wm['Gfold'] = (i128[:, None] % DH
                   == jnp.arange(DH)[None, :]).astype(_f32)
    wm['Pz1'] = jnp.concatenate([jnp.eye(DG, dtype=_f32),
                                 jnp.zeros((DG, DH), _f32)], axis=1)
    wm['Pz2'] = jnp.concatenate([jnp.zeros((DH, DG), _f32),
                                 jnp.eye(DH, dtype=_f32)], axis=1)
    wm['hW1T'] = p['hW1'].T
    wm['hb1'] = p['hb1']
    wm['hW2T'] = p['hW2'].T
    wm['hb2'] = p['hb2']
    wm['hW3T'] = p['hW3'].T
    wm['hb3'] = p['hb3']
    return wm


def kernel(x, edge_index, desc, params):
    p = params
    src1d = edge_index[0].astype(jnp.int32)
    dst1d = edge_index[1].astype(jnp.int32)
    zerosD = jnp.zeros((N, DW), _f32)
    onesD = jnp.ones((CHUNK, DW), _f32)
    zerosA = jnp.zeros((N, DGH), _f32)

    deg_parts = _sc_degree(dst1d, zerosD, onesD)
    deg4d = deg_parts.reshape(2, NG, RB, DW)

    m1, dis3d = _tc1(deg4d, x, p['W1'])
    agg1 = _sc_scatter(m1, src1d, dst1d, zerosA, DGH)
    m2 = _tc2(agg1, m1, dis3d, p['W2'])
    agg2 = _sc_scatter(m2, src1d, dst1d, zerosA, DGH)

    dis2d = dis3d.reshape(B, NPG)
    wm = _prep_weights(p)
    out, beta, alpha = _tc3(agg2, m2, dis2d, desc,
                            p['logscale'].reshape(1, K), wm)
    return out, beta, alpha
